# Initial kernel scaffold; baseline (speedup 1.0000x reference)
import jax
import jax.numpy as jnp
from jax.experimental import pallas as pl


def kernel(atom_list, bond_list, atom_degree_list, bond_degree_list, atom_mask, params):
    raise NotImplementedError("write your pallas kernel here")


# speedup vs baseline: 8.3377x; 8.3377x over previous
import functools

import jax
import jax.numpy as jnp
import numpy as np
from jax.experimental import pallas as pl
from jax.experimental.pallas import tpu as pltpu

D = 150
RADIUS = 3
ATOM_F = 39
BOND_F = 10
NFP = 5


def _lrelu(x):
    return jnp.where(x >= 0, x, 0.01 * x)


def _elu(x):
    return jnp.where(x > 0, x, jnp.exp(x) - 1.0)


def _pack_params(params):
    fps = [params["shared"], params["task1"], params["task2"],
           params["gate1"]["fp"], params["gate2"]["fp"]]

    def st(fn):
        return jnp.stack([fn(p) for p in fps])

    pk = {}
    pk["wa"] = st(lambda p: p["atom_fc"]["W"].T)
    pk["ba"] = st(lambda p: p["atom_fc"]["b"])
    pk["wnba"] = st(lambda p: p["neighbor_fc"]["W"][:, :ATOM_F].T)
    pk["wnbb"] = st(lambda p: p["neighbor_fc"]["W"][:, ATOM_F:].T)
    pk["bnb"] = st(lambda p: p["neighbor_fc"]["b"])

    def str_(fn):
        return jnp.stack([fn(p, r) for p in fps for r in range(RADIUS)])

    pk["al_wa"] = str_(lambda p, r: p["align"][r]["W"][0, :D])
    pk["al_wn"] = str_(lambda p, r: p["align"][r]["W"][0, D:])
    pk["wat"] = str_(lambda p, r: p["attend"][r]["W"].T)
    pk["bat"] = str_(lambda p, r: p["attend"][r]["b"])
    pk["gwi"] = str_(lambda p, r: p["gru"][r]["Wih"].T)
    pk["gwh"] = str_(lambda p, r: p["gru"][r]["Whh"].T)
    pk["gbi"] = str_(lambda p, r: p["gru"][r]["bih"])
    pk["gbh"] = str_(lambda p, r: p["gru"][r]["bhh"])

    pk["mwa"] = st(lambda p: p["mol_align"]["W"][0, :D])
    pk["mwn"] = st(lambda p: p["mol_align"]["W"][0, D:])
    pk["mwat"] = st(lambda p: p["mol_attend"]["W"].T)
    pk["mbat"] = st(lambda p: p["mol_attend"]["b"])
    pk["mgwi"] = st(lambda p: p["mol_gru"]["Wih"].T)
    pk["mgwh"] = st(lambda p: p["mol_gru"]["Whh"].T)
    pk["mgbi"] = st(lambda p: p["mol_gru"]["bih"])
    pk["mgbh"] = st(lambda p: p["mol_gru"]["bhh"])

    pk["gdw"] = jnp.stack([params["gate1"]["dnn"]["W"].T,
                           params["gate2"]["dnn"]["W"].T])
    pk["tw1"] = jnp.stack([params["tower1"]["fc1"]["W"].T,
                           params["tower2"]["fc1"]["W"].T])
    pk["tw2"] = jnp.stack([params["tower1"]["fc2"]["W"].T,
                           params["tower2"]["fc2"]["W"].T])
    pk["tb1"] = jnp.stack([params["tower1"]["fc1"]["b"],
                           params["tower2"]["fc1"]["b"]])

    bank = np.zeros((8, 128), dtype=np.float32)
    sb = jnp.zeros((8, 128), dtype=jnp.float32)
    al_b = jnp.stack([p["align"][r]["b"][0] for p in fps for r in range(RADIUS)])
    sb = sb.at[0, :15].set(al_b)
    sb = sb.at[1, :5].set(jnp.stack([p["mol_align"]["b"][0] for p in fps]))
    sb = sb.at[2, :2].set(params["gate1"]["dnn"]["b"])
    sb = sb.at[2, 2:4].set(params["gate2"]["dnn"]["b"])
    sb = sb.at[3, 0].set(params["tower1"]["fc2"]["b"][0])
    sb = sb.at[3, 1].set(params["tower2"]["fc2"]["b"][0])
    pk["sbank"] = sb
    del bank
    return pk


_WEIGHT_KEYS = ["wa", "ba", "wnba", "wnbb", "bnb", "al_wa", "al_wn", "wat",
                "bat", "gwi", "gwh", "gbi", "gbh", "mwa", "mwn", "mwat",
                "mbat", "mgwi", "mgwh", "mgbi", "mgbh", "gdw", "tw1", "tw2",
                "tb1", "sbank"]


def _gru_step(x, h, wi, wh, bi, bh):
    gi = jnp.dot(x, wi, preferred_element_type=jnp.float32) + bi
    gh = jnp.dot(h, wh, preferred_element_type=jnp.float32) + bh
    r = jax.nn.sigmoid(gi[:, 0:D] + gh[:, 0:D])
    z = jax.nn.sigmoid(gi[:, D:2 * D] + gh[:, D:2 * D])
    n = jnp.tanh(gi[:, 2 * D:3 * D] + r * gh[:, 2 * D:3 * D])
    return (1.0 - z) * n + z * h


def _matic_kernel(L, NB, M,
                  atoms_ref, bonds_ref, adeg_ref, bdeg_ref, mask_ref,
                  wa, ba, wnba, wnbb, bnb, al_wa, al_wn, wat, bat,
                  gwi, gwh, gbi, gbh, mwa, mwn, mwat, mbat,
                  mgwi, mgwh, mgbi, mgbh, gdw, tw1, tw2, tb1, sbank,
                  out_ref, satt_ref, t1att_ref, t2att_ref, sel1_ref, sel2_ref,
                  sf1_ref, t1f1_ref, t2f1_ref, sf2_ref, t1f2_ref, t2f2_ref):
    f32 = jnp.float32
    atoms = atoms_ref[0]
    bonds = bonds_ref[0]
    adeg = adeg_ref[0]
    bdeg = bdeg_ref[0]
    mask = mask_ref[0]

    iota_a = jax.lax.broadcasted_iota(jnp.int32, (L, L), 1)
    iota_b = jax.lax.broadcasted_iota(jnp.int32, (L, NB), 1)
    Ga = jnp.concatenate(
        [(adeg[:, m:m + 1] == iota_a).astype(f32) for m in range(M)], axis=0)
    Gb = jnp.concatenate(
        [(bdeg[:, m:m + 1] == iota_b).astype(f32) for m in range(M)], axis=0)
    pad_hit = jnp.concatenate(
        [(adeg[:, m:m + 1] == L - 1) for m in range(M)], axis=0)
    attend_mask = jnp.where(pad_hit, 0.0, 1.0)
    smask = jnp.where(pad_hit, -9e8, 0.0)

    rawa = jnp.dot(Ga, atoms, preferred_element_type=f32)
    rawb = jnp.dot(Gb, bonds, preferred_element_type=f32)

    mol_smask = jnp.where(mask == 0.0, -9e8, 0.0)

    def softmax_m(sc):
        chunks = [sc[m * L:(m + 1) * L] for m in range(M)]
        mx = functools.reduce(jnp.maximum, chunks)
        es = [jnp.exp(c - mx) for c in chunks]
        tot = functools.reduce(lambda a, b: a + b, es)
        return [e / tot for e in es]

    mol_feats = []
    maws = []
    acts = []
    hs = []
    for f in range(NFP):
        af = _lrelu(jnp.dot(atoms, wa[f], preferred_element_type=f32) +
                    ba[f:f + 1, :])
        nbf = _lrelu(jnp.dot(rawa, wnba[f], preferred_element_type=f32) +
                     jnp.dot(rawb, wnbb[f], preferred_element_type=f32) +
                     bnb[f:f + 1, :])
        h = af
        act = None
        for d in range(RADIUS):
            k = f * RADIUS + d
            al_b = sbank[0:1, k:k + 1]
            afs = jnp.sum(h * al_wa[k:k + 1, :], axis=1, keepdims=True) + al_b
            if d == 0:
                nbs = jnp.sum(nbf * al_wn[k:k + 1, :], axis=1, keepdims=True)
            else:
                pa = jnp.sum(act * al_wn[k:k + 1, :], axis=1, keepdims=True)
                nbs = jnp.dot(Ga, pa, preferred_element_type=f32)
            sc = _lrelu(jnp.concatenate([afs] * M, axis=0) + nbs) + smask
            ws = [w * attend_mask[m * L:(m + 1) * L]
                  for m, w in enumerate(softmax_m(sc))]
            wsum = functools.reduce(lambda a, b: a + b, ws)
            if d == 0:
                ctx_pre = functools.reduce(
                    lambda a, b: a + b,
                    [ws[m] * nbf[m * L:(m + 1) * L] for m in range(M)])
            else:
                S = functools.reduce(
                    lambda a, b: a + b,
                    [ws[m] * Ga[m * L:(m + 1) * L] for m in range(M)])
                ctx_pre = jnp.dot(S, act, preferred_element_type=f32)
            ctx = _elu(jnp.dot(ctx_pre, wat[k], preferred_element_type=f32) +
                       wsum * bat[k:k + 1, :])
            h = _gru_step(ctx, h, gwi[k], gwh[k], gbi[k:k + 1, :], gbh[k:k + 1, :])
            act = jnp.maximum(h, 0.0)

        mol_feature = jnp.sum(act * mask, axis=0, keepdims=True)
        act_mol = jnp.maximum(mol_feature, 0.0)
        mb = sbank[1:2, f:f + 1]
        c1 = jnp.sum(act_mol * mwa[f:f + 1, :], axis=1, keepdims=True) + mb
        s2 = jnp.sum(act * mwn[f:f + 1, :], axis=1, keepdims=True)
        mas = _lrelu(c1 + s2) + mol_smask
        mmx = jnp.max(mas, axis=0, keepdims=True)
        me = jnp.exp(mas - mmx)
        maw = me / jnp.sum(me, axis=0, keepdims=True) * mask
        mol_ctx_pre = jnp.sum(maw * act, axis=0, keepdims=True)
        mwsum = jnp.sum(maw, axis=0, keepdims=True)
        mol_ctx = _elu(jnp.dot(mol_ctx_pre, mwat[f], preferred_element_type=f32) +
                       mwsum * mbat[f:f + 1, :])
        mol_feature = _gru_step(mol_ctx, mol_feature, mgwi[f], mgwh[f],
                                mgbi[f:f + 1, :], mgbh[f:f + 1, :])
        mol_feats.append(mol_feature)
        maws.append(maw)
        acts.append(act)
        hs.append(h)

    sels = []
    for g in range(2):
        logits = jnp.dot(mol_feats[3 + g], gdw[g], preferred_element_type=f32) \
            + sbank[2:3, 2 * g:2 * g + 2]
        mx = jnp.max(logits, axis=1, keepdims=True)
        e = jnp.exp(logits - mx)
        sels.append(e / jnp.sum(e, axis=1, keepdims=True))
    outs = []
    for g in range(2):
        gate_out = sels[g][:, 0:1] * mol_feats[1 + g] + sels[g][:, 1:2] * mol_feats[0]
        hdn = jnp.maximum(jnp.dot(gate_out, tw1[g], preferred_element_type=f32) +
                          tb1[g:g + 1, :], 0.0)
        outs.append(jnp.dot(hdn, tw2[g], preferred_element_type=f32) +
                    sbank[3:4, g:g + 1])

    out_ref[0] = jnp.concatenate(outs, axis=1)
    sel1_ref[0] = sels[0]
    sel2_ref[0] = sels[1]
    satt_ref[0] = maws[0]
    t1att_ref[0] = maws[1]
    t2att_ref[0] = maws[2]
    sf1_ref[0] = acts[0]
    t1f1_ref[0] = acts[1]
    t2f1_ref[0] = acts[2]
    sf2_ref[0] = hs[0]
    t1f2_ref[0] = hs[1]
    t2f2_ref[0] = hs[2]


def kernel(atom_list, bond_list, atom_degree_list, bond_degree_list, atom_mask,
           params, interpret=False):
    B, L, _ = atom_list.shape
    NB = bond_list.shape[1]
    M = atom_degree_list.shape[-1]
    pk = _pack_params(params)

    adeg = atom_degree_list.astype(jnp.int32)
    bdeg = bond_degree_list.astype(jnp.int32)
    mask3 = atom_mask.reshape(B, L, 1)

    def full_spec(arr):
        r = arr.ndim
        return pl.BlockSpec(arr.shape, lambda i, _r=r: (0,) * _r)

    in_specs = [
        pl.BlockSpec((1, L, ATOM_F), lambda i: (i, 0, 0)),
        pl.BlockSpec((1, NB, BOND_F), lambda i: (i, 0, 0)),
        pl.BlockSpec((1, L, M), lambda i: (i, 0, 0)),
        pl.BlockSpec((1, L, M), lambda i: (i, 0, 0)),
        pl.BlockSpec((1, L, 1), lambda i: (i, 0, 0)),
    ] + [full_spec(pk[k]) for k in _WEIGHT_KEYS]

    def o3(shape):
        return (jax.ShapeDtypeStruct((B,) + shape, jnp.float32),
                pl.BlockSpec((1,) + shape, lambda i: (i, 0, 0)))

    out_shapes, out_specs = zip(*[
        o3((1, 2)),
        o3((L, 1)),
        o3((L, 1)),
        o3((L, 1)),
        o3((1, 2)),
        o3((1, 2)),
        o3((L, D)),
        o3((L, D)),
        o3((L, D)),
        o3((L, D)),
        o3((L, D)),
        o3((L, D)),
    ])

    fn = pl.pallas_call(
        functools.partial(_matic_kernel, L, NB, M),
        grid=(B,),
        in_specs=list(in_specs),
        out_specs=list(out_specs),
        out_shape=list(out_shapes),
        interpret=interpret,
    )
    res = fn(atom_list, bond_list, adeg, bdeg, mask3,
             *[pk[k] for k in _WEIGHT_KEYS])
    (out, satt, t1att, t2att, sel1, sel2,
     sf1, t1f1, t2f1, sf2, t1f2, t2f2) = res
    out = out.reshape(B, 2)
    sel1 = sel1.reshape(B, 2)
    sel2 = sel2.reshape(B, 2)
    return (out, [satt, t1att, t2att, sel1, sel2],
            [sf1, t1f1, t2f1], [sf2, t1f2, t2f2])
